# baseline (speedup 1.0000x reference)
import functools

import jax
import jax.numpy as jnp
from jax import lax
from jax.experimental import pallas as pl
from jax.experimental.pallas import tpu as pltpu
from jax.experimental.pallas import tpu_sc as plsc

TIMESTEPS = 1000
EMBED_DIM = 128
BATCH = 16384

_NC = 2
_NS = 16
_NW = _NC * _NS
_BPW = BATCH // _NW
_CHUNK = 64
_NCHUNK = _BPW // _CHUNK
_TROWS = 64


@functools.partial(
    pl.kernel,
    mesh=plsc.VectorSubcoreMesh(core_axis_name="c", subcore_axis_name="s"),
    out_type=jax.ShapeDtypeStruct((BATCH, EMBED_DIM), jnp.float32),
    scratch_types=[
        pltpu.VMEM((_BPW,), jnp.int32),
        pltpu.VMEM((_BPW, EMBED_DIM), jnp.float32),
        pltpu.VMEM_SHARED((TIMESTEPS, EMBED_DIM), jnp.float32),
        pltpu.SemaphoreType.DMA,
        pltpu.SemaphoreType.DMA,
        pltpu.SemaphoreType.DMA,
        pltpu.SemaphoreType.DMA,
        pltpu.SemaphoreType.DMA,
        pltpu.SemaphoreType.DMA,
        pltpu.SemaphoreType.DMA,
        pltpu.SemaphoreType.DMA,
        pltpu.SemaphoreType.DMA,
    ],
)
def _gather_kernel(table_hbm, idx_hbm, out_hbm, idx_v, rows_v, tbl_s,
                   g0, g1, g2, g3, g4, g5, g6, g7, wb_sem):
    gsems = [g0, g1, g2, g3, g4, g5, g6, g7]
    sid = lax.axis_index("s")
    wid = sid * _NC + lax.axis_index("c")
    base = wid * _BPW
    idx_cp = pltpu.async_copy(idx_hbm.at[pl.ds(base, _BPW)], idx_v, wb_sem)
    toff = pl.multiple_of(sid * _TROWS, _TROWS)
    @pl.when(sid < _NS - 1)
    def _():
        pltpu.sync_copy(
            table_hbm.at[pl.ds(toff, _TROWS)],
            tbl_s.at[pl.ds(toff, _TROWS)],
        )
    @pl.when(sid == _NS - 1)
    def _():
        last = (_NS - 1) * _TROWS
        pltpu.sync_copy(
            table_hbm.at[pl.ds(last, TIMESTEPS - last)],
            tbl_s.at[pl.ds(last, TIMESTEPS - last)],
        )
    idx_cp.wait()
    gathers = [
        pltpu.async_copy(
            table_hbm.at[idx_v.at[pl.ds(j * _CHUNK, _CHUNK)]],
            rows_v.at[pl.ds(j * _CHUNK, _CHUNK)],
            gsems[j],
        )
        for j in range(2)
    ]
    plsc.subcore_barrier()
    for j in range(2, _NCHUNK):
        gathers.append(
            pltpu.async_copy(
                tbl_s.at[idx_v.at[pl.ds(j * _CHUNK, _CHUNK)]],
                rows_v.at[pl.ds(j * _CHUNK, _CHUNK)],
                gsems[j],
            )
        )
    writebacks = []
    for j in range(_NCHUNK):
        gathers[j].wait()
        writebacks.append(
            pltpu.async_copy(
                rows_v.at[pl.ds(j * _CHUNK, _CHUNK)],
                out_hbm.at[pl.ds(base + j * _CHUNK, _CHUNK)],
                wb_sem,
            )
        )
    for wb in writebacks:
        wb.wait()


def kernel(t, embed_table):
    return _gather_kernel(embed_table, t)


# speedup vs baseline: 1.0220x; 1.0220x over previous
import functools

import jax
import jax.numpy as jnp
from jax import lax
from jax.experimental import pallas as pl
from jax.experimental.pallas import tpu as pltpu
from jax.experimental.pallas import tpu_sc as plsc

TIMESTEPS = 1000
EMBED_DIM = 128
BATCH = 16384

_NC = 2
_NS = 16
_NW = _NC * _NS
_BPW = BATCH // _NW
_CHUNK = 64
_NCHUNK = _BPW // _CHUNK
_TROWS = 64


@functools.partial(
    pl.kernel,
    mesh=plsc.VectorSubcoreMesh(core_axis_name="c", subcore_axis_name="s"),
    out_type=jax.ShapeDtypeStruct((BATCH, EMBED_DIM), jnp.float32),
    scratch_types=[
        pltpu.VMEM((_BPW,), jnp.int32),
        pltpu.VMEM((_BPW, EMBED_DIM), jnp.float32),
        pltpu.VMEM_SHARED((TIMESTEPS, EMBED_DIM), jnp.float32),
        pltpu.SemaphoreType.DMA,
        pltpu.SemaphoreType.DMA,
        pltpu.SemaphoreType.DMA,
        pltpu.SemaphoreType.DMA,
        pltpu.SemaphoreType.DMA,
        pltpu.SemaphoreType.DMA,
        pltpu.SemaphoreType.DMA,
        pltpu.SemaphoreType.DMA,
        pltpu.SemaphoreType.DMA,
    ],
)
def _gather_kernel(table_hbm, idx_hbm, out_hbm, idx_v, rows_v, tbl_s,
                   g0, g1, g2, g3, g4, g5, g6, g7, wb_sem):
    gsems = [g0, g1, g2, g3, g4, g5, g6, g7]
    sid = lax.axis_index("s")
    wid = sid * _NC + lax.axis_index("c")
    base = wid * _BPW
    idx_cp = pltpu.async_copy(idx_hbm.at[pl.ds(base, _BPW)], idx_v, wb_sem)
    toff = pl.multiple_of(sid * _TROWS, _TROWS)
    @pl.when(sid < _NS - 1)
    def _():
        pltpu.sync_copy(
            table_hbm.at[pl.ds(toff, _TROWS)],
            tbl_s.at[pl.ds(toff, _TROWS)],
        )
    @pl.when(sid == _NS - 1)
    def _():
        last = (_NS - 1) * _TROWS
        pltpu.sync_copy(
            table_hbm.at[pl.ds(last, TIMESTEPS - last)],
            tbl_s.at[pl.ds(last, TIMESTEPS - last)],
        )
    idx_cp.wait()
    gathers = [
        pltpu.async_copy(
            table_hbm.at[idx_v.at[pl.ds(0, _CHUNK)]],
            rows_v.at[pl.ds(0, _CHUNK)],
            gsems[0],
        )
    ]
    plsc.subcore_barrier()
    for j in range(1, _NCHUNK):
        gathers.append(
            pltpu.async_copy(
                tbl_s.at[idx_v.at[pl.ds(j * _CHUNK, _CHUNK)]],
                rows_v.at[pl.ds(j * _CHUNK, _CHUNK)],
                gsems[j],
            )
        )
    writebacks = []
    for j in range(_NCHUNK):
        gathers[j].wait()
        writebacks.append(
            pltpu.async_copy(
                rows_v.at[pl.ds(j * _CHUNK, _CHUNK)],
                out_hbm.at[pl.ds(base + j * _CHUNK, _CHUNK)],
                wb_sem,
            )
        )
    for wb in writebacks:
        wb.wait()


def kernel(t, embed_table):
    return _gather_kernel(embed_table, t)


# speedup vs baseline: 1.0320x; 1.0097x over previous
import functools

import jax
import jax.numpy as jnp
from jax import lax
from jax.experimental import pallas as pl
from jax.experimental.pallas import tpu as pltpu
from jax.experimental.pallas import tpu_sc as plsc

TIMESTEPS = 1000
EMBED_DIM = 128
BATCH = 16384

_NC = 2
_NS = 16
_NW = _NC * _NS
_BPW = BATCH // _NW
_CHUNK = 64
_NCHUNK = _BPW // _CHUNK
_TROWS = 64


@functools.partial(
    pl.kernel,
    mesh=plsc.VectorSubcoreMesh(core_axis_name="c", subcore_axis_name="s"),
    out_type=jax.ShapeDtypeStruct((BATCH, EMBED_DIM), jnp.float32),
    scratch_types=[
        pltpu.VMEM((_BPW,), jnp.int32),
        pltpu.VMEM((_BPW, EMBED_DIM), jnp.float32),
        pltpu.VMEM_SHARED((TIMESTEPS, EMBED_DIM), jnp.float32),
        pltpu.SemaphoreType.DMA,
        pltpu.SemaphoreType.DMA,
        pltpu.SemaphoreType.DMA,
        pltpu.SemaphoreType.DMA,
        pltpu.SemaphoreType.DMA,
        pltpu.SemaphoreType.DMA,
        pltpu.SemaphoreType.DMA,
        pltpu.SemaphoreType.DMA,
        pltpu.SemaphoreType.DMA,
    ],
)
def _gather_kernel(table_hbm, idx_hbm, out_hbm, idx_v, rows_v, tbl_s,
                   g0, g1, g2, g3, g4, g5, g6, g7, wb_sem):
    gsems = [g0, g1, g2, g3, g4, g5, g6, g7]
    sid = lax.axis_index("s")
    wid = sid * _NC + lax.axis_index("c")
    base = wid * _BPW
    idx_cp = pltpu.async_copy(idx_hbm.at[pl.ds(base, _BPW)], idx_v, wb_sem)
    toff = pl.multiple_of(sid * _TROWS, _TROWS)
    @pl.when(sid < _NS - 1)
    def _():
        pltpu.sync_copy(
            table_hbm.at[pl.ds(toff, _TROWS)],
            tbl_s.at[pl.ds(toff, _TROWS)],
        )
    @pl.when(sid == _NS - 1)
    def _():
        last = (_NS - 1) * _TROWS
        pltpu.sync_copy(
            table_hbm.at[pl.ds(last, TIMESTEPS - last)],
            tbl_s.at[pl.ds(last, TIMESTEPS - last)],
        )
    idx_cp.wait()
    gathers = [
        pltpu.async_copy(
            table_hbm.at[idx_v.at[pl.ds(0, _CHUNK)]],
            rows_v.at[pl.ds(0, _CHUNK)],
            gsems[0],
        )
    ]
    plsc.subcore_barrier()
    for j in range(1, _NCHUNK):
        gathers.append(
            pltpu.async_copy(
                tbl_s.at[idx_v.at[pl.ds(j * _CHUNK, _CHUNK)]],
                rows_v.at[pl.ds(j * _CHUNK, _CHUNK)],
                gsems[j],
            )
        )
    writebacks = []
    for j in list(range(1, _NCHUNK)) + [0]:
        gathers[j].wait()
        writebacks.append(
            pltpu.async_copy(
                rows_v.at[pl.ds(j * _CHUNK, _CHUNK)],
                out_hbm.at[pl.ds(base + j * _CHUNK, _CHUNK)],
                wb_sem,
            )
        )
    for wb in writebacks:
        wb.wait()


def kernel(t, embed_table):
    return _gather_kernel(embed_table, t)
